# Initial kernel scaffold; baseline (speedup 1.0000x reference)
import jax
import jax.numpy as jnp
from jax.experimental import pallas as pl


def kernel(c0, cond1, cond4, cond5, emb1):
    raise NotImplementedError("write your pallas kernel here")


# speedup vs baseline: 2.0747x; 2.0747x over previous
import jax
import jax.numpy as jnp
from jax.experimental import pallas as pl
from jax.experimental.pallas import tpu as pltpu

B = 16384
D = 1024
BR = 512


def _body(cond1_ref, emb_ref, c0_ref, cond4_ref, cond5_ref, out_ref):
    f = cond1_ref[...].astype(jnp.float32)
    e0 = emb_ref[0:1, :]
    e1 = emb_ref[1:2, :]
    e = e0 + f * (e1 - e0)
    meta = jnp.concatenate([cond4_ref[...], cond5_ref[...]], axis=1)
    out_ref[...] = c0_ref[...] + e + meta


def kernel(c0, cond1, cond4, cond5, emb1):
    cond1_2d = cond1.reshape(B, 1)
    grid = (B // BR,)
    return pl.pallas_call(
        _body,
        grid=grid,
        in_specs=[
            pl.BlockSpec((BR, 1), lambda i: (i, 0)),
            pl.BlockSpec((2, D), lambda i: (0, 0)),
            pl.BlockSpec((BR, D), lambda i: (i, 0)),
            pl.BlockSpec((BR, D // 2), lambda i: (i, 0)),
            pl.BlockSpec((BR, D // 2), lambda i: (i, 0)),
        ],
        out_specs=pl.BlockSpec((BR, D), lambda i: (i, 0)),
        out_shape=jax.ShapeDtypeStruct((B, D), jnp.float32),
    )(cond1_2d, emb1, c0, cond4, cond5)
